# baseline (speedup 1.0000x reference)
import functools

import jax
import jax.numpy as jnp
from jax import lax
from jax.experimental import pallas as pl
from jax.experimental.pallas import tpu as pltpu
from jax.experimental.pallas import tpu_sc as plsc

N = 160000
C = 256
O = 256

_NW = 32
_RPW = 1920
_NSC = _RPW * _NW
_NTC = N - _NSC
_TCB = 1280
_TCG = _NTC // _TCB
_CH = 192
_NCH = 10
_GPC = _CH // 16
_L = 16

_mesh = plsc.VectorSubcoreMesh(core_axis_name="c", subcore_axis_name="s")


@functools.partial(
    pl.kernel,
    out_type=jax.ShapeDtypeStruct((_NW * 3 * O * _L,), jnp.float32),
    mesh=_mesh,
    compiler_params=pltpu.CompilerParams(use_tc_tiling_on_sc=True),
    scratch_types=[
        pltpu.VMEM((_CH, C), jnp.float32),
        pltpu.VMEM((_CH, C), jnp.float32),
        pltpu.VMEM((_RPW + _L,), jnp.int32),
        pltpu.VMEM((3 * O * _L,), jnp.float32),
        pltpu.VMEM((3 * _L,), jnp.float32),
        pltpu.VMEM((_L,), jnp.int32),
        pltpu.SemaphoreType.DMA,
        pltpu.SemaphoreType.DMA,
    ],
)
def _sc_stage1(
    pred_hbm, t_hbm, out_hbm, xb0, xb1, t_v, acc_v, va_ref, cur_ref, sem0, sem1
):
    wid = lax.axis_index("s") * 2 + lax.axis_index("c")
    base = wid * _RPW

    pltpu.sync_copy(t_hbm.at[pl.ds(base, _RPW)], t_v.at[pl.ds(0, _RPW)])

    zero16 = jnp.zeros((_L,), jnp.float32)

    def zbody(i, carry):
        acc_v[pl.ds(i * _L, _L)] = zero16
        return carry

    lax.fori_loop(0, 3 * O, zbody, 0)

    lane_inc = jnp.full((_L,), 1.0 / float(_L), jnp.float32)

    def chunk_src(c):
        return pred_hbm.at[pl.ds(base + c * _CH, _CH), :]

    pltpu.async_copy(chunk_src(0), xb0, sem0)
    pltpu.async_copy(chunk_src(1), xb1, sem1)

    def flush(cur, va1, va2, vcn):
        plsc.addupdate(acc_v.at[pl.ds(cur * _L, _L)], va1)
        plsc.addupdate(acc_v.at[pl.ds((cur + O) * _L, _L)], va2)
        plsc.addupdate(acc_v.at[pl.ds((cur + 2 * O) * _L, _L)], vcn)

    def _tree(vals):
        while len(vals) > 1:
            vals = [a + b for a, b in zip(vals[::2], vals[1::2])]
        return vals[0]

    def accum_row(buf, r, va1, va2):
        xs = [buf[r, pl.ds(kk * _L, _L)] for kk in range(C // _L)]
        va1 = va1 + _tree(xs)
        va2 = va2 + _tree([x * x for x in xs])
        return va1, va2

    va_ref[pl.ds(0, _L)] = zero16
    va_ref[pl.ds(_L, _L)] = zero16
    va_ref[pl.ds(2 * _L, _L)] = zero16
    cur_ref[pl.ds(0, _L)] = jnp.zeros((_L,), jnp.int32)

    def slow_rows(buf, row0, tv, nrows, lane0):
        va1 = va_ref[pl.ds(0, _L)]
        va2 = va_ref[pl.ds(_L, _L)]
        vcn = va_ref[pl.ds(2 * _L, _L)]
        cur = cur_ref[pl.ds(0, _L)][0]
        for rr in range(nrows):
            t_r = tv[lane0 + rr]
            changed = t_r != cur

            @pl.when(changed)
            def _():
                flush(cur, va1, va2, vcn)

            va1 = jnp.where(changed, 0.0, va1)
            va2 = jnp.where(changed, 0.0, va2)
            vcn = jnp.where(changed, 0.0, vcn)
            cur = jnp.where(changed, t_r, cur)
            va1, va2 = accum_row(buf, row0 + rr, va1, va2)
            vcn = vcn + lane_inc
        va_ref[pl.ds(0, _L)] = va1
        va_ref[pl.ds(_L, _L)] = va2
        va_ref[pl.ds(2 * _L, _L)] = vcn
        cur_ref[pl.ds(0, _L)] = jnp.broadcast_to(cur, (_L,))

    def make_group(buf):
        def group(c, g):
            tv = t_v[pl.ds(c * _CH + g * _L, _L)]
            cur = cur_ref[pl.ds(0, _L)][0]
            uniform = jnp.logical_and(tv[0] == tv[_L - 1], tv[0] == cur)

            @pl.when(uniform)
            def _fast():
                va1 = va_ref[pl.ds(0, _L)]
                va2 = va_ref[pl.ds(_L, _L)]
                for rr in range(_L):
                    va1, va2 = accum_row(buf, g * _L + rr, va1, va2)
                va_ref[pl.ds(0, _L)] = va1
                va_ref[pl.ds(_L, _L)] = va2
                va_ref[pl.ds(2 * _L, _L)] = va_ref[pl.ds(2 * _L, _L)] + 1.0

            @pl.when(jnp.logical_not(uniform))
            def _slow():
                slow_rows(buf, g * _L, tv, _L, 0)

        return group

    group0 = make_group(xb0)
    group1 = make_group(xb1)

    def pair(j, carry):
        c0 = 2 * j
        pltpu.make_async_copy(chunk_src(c0), xb0, sem0).wait()

        def g0(g, a):
            group0(c0, g)
            return a

        lax.fori_loop(0, _GPC, g0, 0)

        @pl.when(j < (_NCH // 2) - 1)
        def _():
            pltpu.async_copy(chunk_src(c0 + 2), xb0, sem0)

        c1 = 2 * j + 1
        pltpu.make_async_copy(chunk_src(c1), xb1, sem1).wait()

        def g1(g, a):
            group1(c1, g)
            return a

        lax.fori_loop(0, _GPC, g1, 0)

        @pl.when(j < (_NCH // 2) - 1)
        def _():
            pltpu.async_copy(chunk_src(c1 + 2), xb1, sem1)

        return carry

    lax.fori_loop(0, _NCH // 2, pair, 0)

    flush(
        cur_ref[pl.ds(0, _L)][0],
        va_ref[pl.ds(0, _L)],
        va_ref[pl.ds(_L, _L)],
        va_ref[pl.ds(2 * _L, _L)],
    )
    pltpu.sync_copy(acc_v, out_hbm.at[pl.ds(wid * 3 * O * _L, 3 * O * _L)])


def _tc_partial(t_ref, x_ref, out_ref):
    i = pl.program_id(0)

    @pl.when(i == 0)
    def _init():
        out_ref[...] = jnp.zeros_like(out_ref)

    x = x_ref[...]
    tb = t_ref[0]
    col = lax.broadcasted_iota(jnp.int32, (1, C), 1)
    ohf = (tb == col).astype(jnp.float32)
    ones_c1 = jnp.ones((C, 1), jnp.float32)
    rs1 = jnp.dot(x, ones_c1)
    rq1 = jnp.dot(x * x, ones_c1)
    y = jnp.concatenate(
        [rs1, rq1, jnp.ones((_TCB, 1), jnp.float32),
         jnp.zeros((_TCB, 5), jnp.float32)], axis=1)
    part = lax.dot_general(
        ohf, y, (((0,), (0,)), ((), ())),
        preferred_element_type=jnp.float32)
    out_ref[...] += part


def _stage2(p_ref, tcp_ref, out_ref, n_ref, a_ref, sb_ref, st_ref):
    x = p_ref[...]
    s = jnp.sum(jnp.sum(x, axis=3), axis=0)
    tcT = tcp_ref[:, 0:3].T
    s1 = s[0:1, :] + tcT[0:1, :]
    s2 = s[1:2, :] + tcT[1:2, :]
    cnt = s[2:3, :] + tcT[2:3, :]
    validr = lax.broadcasted_iota(jnp.int32, (1, O), 1) < (O - 1)
    ne = cnt * float(C)
    mean = jnp.where(validr, s1 / jnp.maximum(ne, 1.0), 0.0)
    var = (s2 - ne * mean * mean) / jnp.maximum(ne - 1.0, 1.0)
    stdr = jnp.sqrt(jnp.maximum(var, 0.0))
    stdr = jnp.where(jnp.logical_and(cnt > 1.0, validr), stdr, 0.0)
    ms = jnp.concatenate([mean, stdr], axis=0)
    st_ref[:, 0:2] = ms.T
    cols = lax.broadcasted_iota(jnp.int32, (32, O), 1)

    def init_rb(rb, carry):
        mT = st_ref[pl.ds(rb * 32, 32), 0:1]
        d = mean - mT
        ad = jnp.abs(d)
        n = 16.0 * ad
        n_ref[pl.ds(rb * 32, 32), :] = 2.0 * n + 1e-5
        a_ref[pl.ds(rb * 32, 32), :] = ad / (n + 1e-5)
        sT = st_ref[pl.ds(rb * 32, 32), 1:2]
        sb_ref[pl.ds(rb * 32, 32), :] = jnp.broadcast_to(sT, (32, O))
        return carry

    lax.fori_loop(0, 8, init_rb, 0)

    def outer(rb, tot):
        ab = a_ref[pl.ds(rb * 32, 32), :]
        eb = ab * stdr + (1.0 + 1e-5)

        def inner(c4, acc):
            for u in range(4):
                c = c4 * 4 + u
                num = n_ref[pl.ds(c, 1), :]
                srow = sb_ref[pl.ds(c, 1), :]
                nrow = (num - 1e-5) * 0.5
                acc = acc + num / (eb + nrow + ab * srow)
            return acc

        acc = lax.fori_loop(0, O // 4, inner, jnp.zeros((32, O), jnp.float32))
        M = 1.0 - acc * (1.0 / float(C))
        rows = lax.broadcasted_iota(jnp.int32, (32, O), 0) + rb * 32
        sel = rows > cols
        val = -M * jnp.log(1.0 - M)
        return tot + jnp.sum(jnp.where(sel, val, 0.0))

    tot = lax.fori_loop(0, 8, outer, jnp.zeros((), jnp.float32))
    out_ref[0, 0] = tot * (2.0 / float(O * (O - 1)))


@jax.jit
def kernel(pred_id, target_id):
    t = target_id.astype(jnp.int32)
    partials = _sc_stage1(pred_id, t)
    t3 = lax.slice(t, (_NSC,), (N,)).reshape(_TCG, _TCB, 1)
    tc_part = pl.pallas_call(
        _tc_partial,
        grid=(_TCG,),
        in_specs=[
            pl.BlockSpec((1, _TCB, 1), lambda i: (i, 0, 0)),
            pl.BlockSpec((_TCB, C), lambda i: (i + 48, 0)),
        ],
        out_specs=pl.BlockSpec((C, 8), lambda i: (0, 0)),
        out_shape=jax.ShapeDtypeStruct((C, 8), jnp.float32),
    )(t3, pred_id)
    loss = pl.pallas_call(
        _stage2,
        in_specs=[
            pl.BlockSpec(memory_space=pltpu.VMEM),
            pl.BlockSpec(memory_space=pltpu.VMEM),
        ],
        out_specs=pl.BlockSpec(memory_space=pltpu.SMEM),
        out_shape=jax.ShapeDtypeStruct((1, 1), jnp.float32),
        scratch_shapes=[
            pltpu.VMEM((O, O), jnp.float32),
            pltpu.VMEM((O, O), jnp.float32),
            pltpu.VMEM((O, O), jnp.float32),
            pltpu.VMEM((O, 8), jnp.float32),
        ],
    )(partials.reshape(_NW, 3, O, _L), tc_part)
    return loss[0, 0]


# speedup vs baseline: 2.1520x; 2.1520x over previous
import functools

import jax
import jax.numpy as jnp
from jax import lax
from jax.experimental import pallas as pl
from jax.experimental.pallas import tpu as pltpu
from jax.experimental.pallas import tpu_sc as plsc

N = 160000
C = 256
O = 256

_NW = 32
_RPW = N // _NW
_CH = 192
_NCH = 26
_TAIL = _RPW - _NCH * _CH
_GPC = _CH // 16
_L = 16

_mesh = plsc.VectorSubcoreMesh(core_axis_name="c", subcore_axis_name="s")


@functools.partial(
    pl.kernel,
    out_type=jax.ShapeDtypeStruct((_NW * 3 * O * _L,), jnp.float32),
    mesh=_mesh,
    compiler_params=pltpu.CompilerParams(use_tc_tiling_on_sc=True),
    scratch_types=[
        pltpu.VMEM((_CH, C), jnp.float32),
        pltpu.VMEM((_CH, C), jnp.float32),
        pltpu.VMEM((_RPW + _L,), jnp.int32),
        pltpu.VMEM((3 * O * _L,), jnp.float32),
        pltpu.VMEM((3 * _L,), jnp.float32),
        pltpu.VMEM((_L,), jnp.int32),
        pltpu.SemaphoreType.DMA,
        pltpu.SemaphoreType.DMA,
    ],
)
def _sc_stage1(
    pred_hbm, t_hbm, out_hbm, xb0, xb1, t_v, acc_v, va_ref, cur_ref, sem0, sem1
):
    wid = lax.axis_index("s") * 2 + lax.axis_index("c")
    base = wid * _RPW

    pltpu.sync_copy(t_hbm.at[pl.ds(base, _RPW)], t_v.at[pl.ds(0, _RPW)])

    zero16 = jnp.zeros((_L,), jnp.float32)

    def zbody(i, carry):
        acc_v[pl.ds(i * _L, _L)] = zero16
        return carry

    lax.fori_loop(0, 3 * O, zbody, 0)

    lane_inc = jnp.full((_L,), 1.0 / float(_L), jnp.float32)

    def chunk_src(c):
        return pred_hbm.at[pl.ds(base + c * _CH, _CH), :]

    pltpu.async_copy(chunk_src(0), xb0, sem0)
    pltpu.async_copy(chunk_src(1), xb1, sem1)

    def flush(cur, va1, va2, vcn):
        plsc.addupdate(acc_v.at[pl.ds(cur * _L, _L)], va1)
        plsc.addupdate(acc_v.at[pl.ds((cur + O) * _L, _L)], va2)
        plsc.addupdate(acc_v.at[pl.ds((cur + 2 * O) * _L, _L)], vcn)

    def _tree(vals):
        while len(vals) > 1:
            vals = [a + b for a, b in zip(vals[::2], vals[1::2])]
        return vals[0]

    def accum_row(buf, r, va1, va2):
        xs = [buf[r, pl.ds(kk * _L, _L)] for kk in range(C // _L)]
        va1 = va1 + _tree(xs)
        va2 = va2 + _tree([x * x for x in xs])
        return va1, va2

    va_ref[pl.ds(0, _L)] = zero16
    va_ref[pl.ds(_L, _L)] = zero16
    va_ref[pl.ds(2 * _L, _L)] = zero16
    cur_ref[pl.ds(0, _L)] = jnp.zeros((_L,), jnp.int32)

    def slow_rows(buf, row0, tv, nrows, lane0):
        va1 = va_ref[pl.ds(0, _L)]
        va2 = va_ref[pl.ds(_L, _L)]
        vcn = va_ref[pl.ds(2 * _L, _L)]
        cur = cur_ref[pl.ds(0, _L)][0]
        for rr in range(nrows):
            t_r = tv[lane0 + rr]
            changed = t_r != cur

            @pl.when(changed)
            def _():
                flush(cur, va1, va2, vcn)

            va1 = jnp.where(changed, 0.0, va1)
            va2 = jnp.where(changed, 0.0, va2)
            vcn = jnp.where(changed, 0.0, vcn)
            cur = jnp.where(changed, t_r, cur)
            va1, va2 = accum_row(buf, row0 + rr, va1, va2)
            vcn = vcn + lane_inc
        va_ref[pl.ds(0, _L)] = va1
        va_ref[pl.ds(_L, _L)] = va2
        va_ref[pl.ds(2 * _L, _L)] = vcn
        cur_ref[pl.ds(0, _L)] = jnp.broadcast_to(cur, (_L,))

    def make_group(buf):
        def group(c, g):
            tv = t_v[pl.ds(c * _CH + g * _L, _L)]
            cur = cur_ref[pl.ds(0, _L)][0]
            uniform = jnp.logical_and(tv[0] == tv[_L - 1], tv[0] == cur)

            @pl.when(uniform)
            def _fast():
                va1 = va_ref[pl.ds(0, _L)]
                va2 = va_ref[pl.ds(_L, _L)]
                for rr in range(_L):
                    va1, va2 = accum_row(buf, g * _L + rr, va1, va2)
                va_ref[pl.ds(0, _L)] = va1
                va_ref[pl.ds(_L, _L)] = va2
                va_ref[pl.ds(2 * _L, _L)] = va_ref[pl.ds(2 * _L, _L)] + 1.0

            @pl.when(jnp.logical_not(uniform))
            def _slow():
                slow_rows(buf, g * _L, tv, _L, 0)

        return group

    group0 = make_group(xb0)
    group1 = make_group(xb1)

    def pair(j, carry):
        c0 = 2 * j
        pltpu.make_async_copy(chunk_src(c0), xb0, sem0).wait()

        def g0(g, a):
            group0(c0, g)
            return a

        lax.fori_loop(0, _GPC, g0, 0)

        @pl.when(j < (_NCH // 2) - 1)
        def _():
            pltpu.async_copy(chunk_src(c0 + 2), xb0, sem0)

        c1 = 2 * j + 1
        pltpu.make_async_copy(chunk_src(c1), xb1, sem1).wait()

        def g1(g, a):
            group1(c1, g)
            return a

        lax.fori_loop(0, _GPC, g1, 0)

        @pl.when(j < (_NCH // 2) - 1)
        def _():
            pltpu.async_copy(chunk_src(c1 + 2), xb1, sem1)

        return carry

    lax.fori_loop(0, _NCH // 2, pair, 0)

    pltpu.sync_copy(
        pred_hbm.at[pl.ds(base + _NCH * _CH, _TAIL), :],
        xb0.at[pl.ds(0, _TAIL), :],
    )
    tvt = t_v[pl.ds(_NCH * _CH, _L)]
    slow_rows(xb0, 0, tvt, _TAIL, 0)
    flush(
        cur_ref[pl.ds(0, _L)][0],
        va_ref[pl.ds(0, _L)],
        va_ref[pl.ds(_L, _L)],
        va_ref[pl.ds(2 * _L, _L)],
    )
    pltpu.sync_copy(acc_v, out_hbm.at[pl.ds(wid * 3 * O * _L, 3 * O * _L)])


def _stage2(p_ref, out_ref, n_ref, a_ref, sb_ref, st_ref):
    x = p_ref[...]
    s = jnp.sum(jnp.sum(x, axis=3), axis=0)
    s1 = s[0:1, :]
    s2 = s[1:2, :]
    cnt = s[2:3, :]
    validr = lax.broadcasted_iota(jnp.int32, (1, O), 1) < (O - 1)
    ne = cnt * float(C)
    mean = jnp.where(validr, s1 / jnp.maximum(ne, 1.0), 0.0)
    var = (s2 - ne * mean * mean) / jnp.maximum(ne - 1.0, 1.0)
    stdr = jnp.sqrt(jnp.maximum(var, 0.0))
    stdr = jnp.where(jnp.logical_and(cnt > 1.0, validr), stdr, 0.0)
    ms = jnp.concatenate([mean, stdr], axis=0)
    st_ref[:, 0:2] = ms.T
    cols = lax.broadcasted_iota(jnp.int32, (32, O), 1)

    def init_rb(rb, carry):
        mT = st_ref[pl.ds(rb * 32, 32), 0:1]
        d = mean - mT
        ad = jnp.abs(d)
        n = 16.0 * ad
        n_ref[pl.ds(rb * 32, 32), :] = 2.0 * n + 1e-5
        a_ref[pl.ds(rb * 32, 32), :] = ad / (n + 1e-5)
        sT = st_ref[pl.ds(rb * 32, 32), 1:2]
        sb_ref[pl.ds(rb * 32, 32), :] = jnp.broadcast_to(sT, (32, O))
        return carry

    lax.fori_loop(0, 8, init_rb, 0)

    def outer(rb, tot):
        ab = a_ref[pl.ds(rb * 32, 32), :]
        eb = ab * stdr + (1.0 + 1e-5)

        def inner(c4, acc):
            for u in range(4):
                c = c4 * 4 + u
                num = n_ref[pl.ds(c, 1), :]
                srow = sb_ref[pl.ds(c, 1), :]
                nrow = (num - 1e-5) * 0.5
                acc = acc + num / (eb + nrow + ab * srow)
            return acc

        acc = lax.fori_loop(0, O // 4, inner, jnp.zeros((32, O), jnp.float32))
        M = 1.0 - acc * (1.0 / float(C))
        rows = lax.broadcasted_iota(jnp.int32, (32, O), 0) + rb * 32
        sel = rows > cols
        val = -M * jnp.log(1.0 - M)
        return tot + jnp.sum(jnp.where(sel, val, 0.0))

    tot = lax.fori_loop(0, 8, outer, jnp.zeros((), jnp.float32))
    out_ref[0, 0] = tot * (2.0 / float(O * (O - 1)))


@jax.jit
def kernel(pred_id, target_id):
    t = target_id.astype(jnp.int32)
    partials = _sc_stage1(pred_id, t)
    loss = pl.pallas_call(
        _stage2,
        in_specs=[pl.BlockSpec(memory_space=pltpu.VMEM)],
        out_specs=pl.BlockSpec(memory_space=pltpu.SMEM),
        out_shape=jax.ShapeDtypeStruct((1, 1), jnp.float32),
        scratch_shapes=[
            pltpu.VMEM((O, O), jnp.float32),
            pltpu.VMEM((O, O), jnp.float32),
            pltpu.VMEM((O, O), jnp.float32),
            pltpu.VMEM((O, 8), jnp.float32),
        ],
    )(partials.reshape(_NW, 3, O, _L))
    return loss[0, 0]


# speedup vs baseline: 2.1780x; 1.0121x over previous
import functools

import jax
import jax.numpy as jnp
from jax import lax
from jax.experimental import pallas as pl
from jax.experimental.pallas import tpu as pltpu
from jax.experimental.pallas import tpu_sc as plsc

N = 160000
C = 256
O = 256

_NW = 32
_RPW = N // _NW
_CH = 192
_NCH = 26
_TAIL = _RPW - _NCH * _CH
_GPC = _CH // 16
_L = 16

_mesh = plsc.VectorSubcoreMesh(core_axis_name="c", subcore_axis_name="s")


@functools.partial(
    pl.kernel,
    out_type=jax.ShapeDtypeStruct((_NW * 3 * O * _L,), jnp.float32),
    mesh=_mesh,
    compiler_params=pltpu.CompilerParams(use_tc_tiling_on_sc=True),
    scratch_types=[
        pltpu.VMEM((_CH, C), jnp.float32),
        pltpu.VMEM((_CH, C), jnp.float32),
        pltpu.VMEM((_RPW + _L,), jnp.int32),
        pltpu.VMEM((3 * O * _L,), jnp.float32),
        pltpu.VMEM((3 * _L,), jnp.float32),
        pltpu.VMEM((_L,), jnp.int32),
        pltpu.SemaphoreType.DMA,
        pltpu.SemaphoreType.DMA,
    ],
)
def _sc_stage1(
    pred_hbm, t_hbm, out_hbm, xb0, xb1, t_v, acc_v, va_ref, cur_ref, sem0, sem1
):
    wid = lax.axis_index("s") * 2 + lax.axis_index("c")
    base = wid * _RPW

    pltpu.sync_copy(t_hbm.at[pl.ds(base, _RPW)], t_v.at[pl.ds(0, _RPW)])

    zero16 = jnp.zeros((_L,), jnp.float32)

    def zbody(i, carry):
        acc_v[pl.ds(i * _L, _L)] = zero16
        return carry

    lax.fori_loop(0, 3 * O, zbody, 0)

    lane_inc = jnp.full((_L,), 1.0 / float(_L), jnp.float32)

    def chunk_src(c):
        return pred_hbm.at[pl.ds(base + c * _CH, _CH), :]

    pltpu.async_copy(chunk_src(0), xb0, sem0)
    pltpu.async_copy(chunk_src(1), xb1, sem1)

    def flush(cur, va1, va2, vcn):
        plsc.addupdate(acc_v.at[pl.ds(cur * _L, _L)], va1)
        plsc.addupdate(acc_v.at[pl.ds((cur + O) * _L, _L)], va2)
        plsc.addupdate(acc_v.at[pl.ds((cur + 2 * O) * _L, _L)], vcn)

    def _tree(vals):
        while len(vals) > 1:
            vals = [a + b for a, b in zip(vals[::2], vals[1::2])]
        return vals[0]

    def accum_row(buf, r, va1, va2):
        xs = [buf[r, pl.ds(kk * _L, _L)] for kk in range(C // _L)]
        va1 = va1 + _tree(xs)
        va2 = va2 + _tree([x * x for x in xs])
        return va1, va2

    va_ref[pl.ds(0, _L)] = zero16
    va_ref[pl.ds(_L, _L)] = zero16
    va_ref[pl.ds(2 * _L, _L)] = zero16
    cur_ref[pl.ds(0, _L)] = jnp.zeros((_L,), jnp.int32)

    def slow_rows(buf, row0, tv, nrows, lane0):
        va1 = va_ref[pl.ds(0, _L)]
        va2 = va_ref[pl.ds(_L, _L)]
        vcn = va_ref[pl.ds(2 * _L, _L)]
        cur = cur_ref[pl.ds(0, _L)][0]
        for rr in range(nrows):
            t_r = tv[lane0 + rr]
            changed = t_r != cur

            @pl.when(changed)
            def _():
                flush(cur, va1, va2, vcn)

            va1 = jnp.where(changed, 0.0, va1)
            va2 = jnp.where(changed, 0.0, va2)
            vcn = jnp.where(changed, 0.0, vcn)
            cur = jnp.where(changed, t_r, cur)
            va1, va2 = accum_row(buf, row0 + rr, va1, va2)
            vcn = vcn + lane_inc
        va_ref[pl.ds(0, _L)] = va1
        va_ref[pl.ds(_L, _L)] = va2
        va_ref[pl.ds(2 * _L, _L)] = vcn
        cur_ref[pl.ds(0, _L)] = jnp.broadcast_to(cur, (_L,))

    def make_group(buf):
        def group(c, g):
            tv = t_v[pl.ds(c * _CH + g * _L, _L)]
            cur = cur_ref[pl.ds(0, _L)][0]
            uniform = jnp.logical_and(tv[0] == tv[_L - 1], tv[0] == cur)

            @pl.when(uniform)
            def _fast():
                va1 = va_ref[pl.ds(0, _L)]
                va2 = va_ref[pl.ds(_L, _L)]
                for rr in range(_L):
                    va1, va2 = accum_row(buf, g * _L + rr, va1, va2)
                va_ref[pl.ds(0, _L)] = va1
                va_ref[pl.ds(_L, _L)] = va2
                va_ref[pl.ds(2 * _L, _L)] = va_ref[pl.ds(2 * _L, _L)] + 1.0

            @pl.when(jnp.logical_not(uniform))
            def _slow():
                slow_rows(buf, g * _L, tv, _L, 0)

        return group

    group0 = make_group(xb0)
    group1 = make_group(xb1)

    def pair(j, carry):
        c0 = 2 * j
        pltpu.make_async_copy(chunk_src(c0), xb0, sem0).wait()

        def g0(g, a):
            group0(c0, g)
            return a

        lax.fori_loop(0, _GPC, g0, 0)

        @pl.when(j < (_NCH // 2) - 1)
        def _():
            pltpu.async_copy(chunk_src(c0 + 2), xb0, sem0)

        c1 = 2 * j + 1
        pltpu.make_async_copy(chunk_src(c1), xb1, sem1).wait()

        def g1(g, a):
            group1(c1, g)
            return a

        lax.fori_loop(0, _GPC, g1, 0)

        @pl.when(j < (_NCH // 2) - 1)
        def _():
            pltpu.async_copy(chunk_src(c1 + 2), xb1, sem1)

        return carry

    lax.fori_loop(0, _NCH // 2, pair, 0)

    pltpu.sync_copy(
        pred_hbm.at[pl.ds(base + _NCH * _CH, _TAIL), :],
        xb0.at[pl.ds(0, _TAIL), :],
    )
    tvt = t_v[pl.ds(_NCH * _CH, _L)]
    slow_rows(xb0, 0, tvt, _TAIL, 0)
    flush(
        cur_ref[pl.ds(0, _L)][0],
        va_ref[pl.ds(0, _L)],
        va_ref[pl.ds(_L, _L)],
        va_ref[pl.ds(2 * _L, _L)],
    )
    pltpu.sync_copy(acc_v, out_hbm.at[pl.ds(wid * 3 * O * _L, 3 * O * _L)])


def _stage2(p_ref, out_ref, n_ref, a_ref, sb_ref, st_ref):
    x = p_ref[...]
    s = jnp.sum(jnp.sum(x, axis=3), axis=0)
    s1 = s[0:1, :]
    s2 = s[1:2, :]
    cnt = s[2:3, :]
    validr = lax.broadcasted_iota(jnp.int32, (1, O), 1) < (O - 1)
    ne = cnt * float(C)
    mean = jnp.where(validr, s1 / jnp.maximum(ne, 1.0), 0.0)
    var = (s2 - ne * mean * mean) / jnp.maximum(ne - 1.0, 1.0)
    stdr = jnp.sqrt(jnp.maximum(var, 0.0))
    stdr = jnp.where(jnp.logical_and(cnt > 1.0, validr), stdr, 0.0)
    ms = jnp.concatenate([mean, stdr], axis=0)
    st_ref[:, 0:2] = ms.T
    cols = lax.broadcasted_iota(jnp.int32, (32, O), 1)

    def init_rb(rb, carry):
        mT = st_ref[pl.ds(rb * 32, 32), 0:1]
        d = mean - mT
        ad = jnp.abs(d)
        n = 16.0 * ad
        n_ref[pl.ds(rb * 32, 32), :] = 2.0 * n + 1e-5
        a_ref[pl.ds(rb * 32, 32), :] = ad / (n + 1e-5)
        sT = st_ref[pl.ds(rb * 32, 32), 1:2]
        sb_ref[pl.ds(rb * 32, 32), :] = jnp.broadcast_to(sT, (32, O))
        return carry

    lax.fori_loop(0, 8, init_rb, 0)

    def outer(rb, tot):
        ab = a_ref[pl.ds(rb * 32, 32), :]
        eb = ab * stdr + (1.0 + 1e-5)

        def inner(c4, acc):
            for u in range(8):
                c = c4 * 8 + u
                num = n_ref[pl.ds(c, 1), :]
                srow = sb_ref[pl.ds(c, 1), :]
                nrow = (num - 1e-5) * 0.5
                acc = acc + num / (eb + nrow + ab * srow)
            return acc

        acc = lax.fori_loop(0, O // 8, inner, jnp.zeros((32, O), jnp.float32))
        M = 1.0 - acc * (1.0 / float(C))
        rows = lax.broadcasted_iota(jnp.int32, (32, O), 0) + rb * 32
        sel = rows > cols
        val = -M * jnp.log(1.0 - M)
        return tot + jnp.sum(jnp.where(sel, val, 0.0))

    tot = lax.fori_loop(0, 8, outer, jnp.zeros((), jnp.float32))
    out_ref[0, 0] = tot * (2.0 / float(O * (O - 1)))


@jax.jit
def kernel(pred_id, target_id):
    t = target_id.astype(jnp.int32)
    partials = _sc_stage1(pred_id, t)
    loss = pl.pallas_call(
        _stage2,
        in_specs=[pl.BlockSpec(memory_space=pltpu.VMEM)],
        out_specs=pl.BlockSpec(memory_space=pltpu.SMEM),
        out_shape=jax.ShapeDtypeStruct((1, 1), jnp.float32),
        scratch_shapes=[
            pltpu.VMEM((O, O), jnp.float32),
            pltpu.VMEM((O, O), jnp.float32),
            pltpu.VMEM((O, O), jnp.float32),
            pltpu.VMEM((O, 8), jnp.float32),
        ],
    )(partials.reshape(_NW, 3, O, _L))
    return loss[0, 0]
